# baseline (speedup 1.0000x reference)
import functools

import jax
import jax.numpy as jnp
from jax import lax
from jax.experimental import pallas as pl
from jax.experimental.pallas import tpu as pltpu
from jax.experimental.pallas import tpu_sc as plsc

N = 10000
E = 320000
D_IN = 128
HID = 128
N_CLS = 16

NPAD = 10240
NC = 2
NS = 16
NW = NC * NS
CH = 128
NCH = 80
EP = NW * NCH * CH
RPS = NPAD // NS


@functools.cache
def _make_deg_kernel():
    mesh = plsc.VectorSubcoreMesh(core_axis_name="c", subcore_axis_name="s")

    @functools.partial(
        pl.kernel,
        mesh=mesh,
        out_type=jax.ShapeDtypeStruct((NC, NPAD, 16), jnp.float32),
        scratch_types=[
            pltpu.VMEM((NCH, CH), jnp.int32),
            pltpu.VMEM((CH, 16), jnp.float32),
            pltpu.VMEM_SHARED((NPAD, 16), jnp.float32),
        ],
        compiler_params=pltpu.CompilerParams(use_tc_tiling_on_sc=False),
    )
    def deg_kernel(dst3d, ones_hbm, zeros_hbm, out_hbm, dst_v, ones_v, acc_s):
        cid = lax.axis_index("c")
        sid = lax.axis_index("s")
        wid = cid * NS + sid
        pltpu.sync_copy(zeros_hbm.at[pl.ds(sid * RPS, RPS)],
                        acc_s.at[pl.ds(sid * RPS, RPS)])
        pltpu.sync_copy(dst3d.at[wid], dst_v)
        pltpu.sync_copy(ones_hbm, ones_v)
        plsc.subcore_barrier()

        def body(j, carry):
            pltpu.sync_copy(ones_v, acc_s.at[dst_v.at[j]], add=True)
            return carry

        lax.fori_loop(0, NCH, body, 0)
        plsc.subcore_barrier()
        pltpu.sync_copy(acc_s.at[pl.ds(sid * RPS, RPS)],
                        out_hbm.at[cid, pl.ds(sid * RPS, RPS)])

    return deg_kernel


@functools.cache
def _make_agg_kernel(D):
    mesh = plsc.VectorSubcoreMesh(core_axis_name="c", subcore_axis_name="s")

    @functools.partial(
        pl.kernel,
        mesh=mesh,
        out_type=jax.ShapeDtypeStruct((NC, NPAD, D), jnp.float32),
        scratch_types=[
            pltpu.VMEM((NCH, CH), jnp.int32),
            pltpu.VMEM((NCH, CH), jnp.int32),
            pltpu.VMEM((CH, D), jnp.float32),
            pltpu.VMEM_SHARED((NPAD, D), jnp.float32),
            pltpu.SemaphoreType.DMA,
        ],
        compiler_params=pltpu.CompilerParams(
            use_tc_tiling_on_sc=(D % 128 == 0)),
    )
    def agg_kernel(hs_hbm, src3d, dst3d, zeros_hbm, out_hbm,
                   src_v, dst_v, rows_v, acc_s, sem):
        cid = lax.axis_index("c")
        sid = lax.axis_index("s")
        wid = cid * NS + sid
        pltpu.sync_copy(zeros_hbm.at[pl.ds(sid * RPS, RPS)],
                        acc_s.at[pl.ds(sid * RPS, RPS)])
        pltpu.sync_copy(src3d.at[wid], src_v)
        pltpu.sync_copy(dst3d.at[wid], dst_v)
        plsc.subcore_barrier()

        def body(j, carry):
            pltpu.async_copy(hs_hbm.at[src_v.at[j]], rows_v, sem).wait()
            pltpu.sync_copy(rows_v, acc_s.at[dst_v.at[j]], add=True)
            return carry

        lax.fori_loop(0, NCH, body, 0)
        plsc.subcore_barrier()
        pltpu.sync_copy(acc_s.at[pl.ds(sid * RPS, RPS)],
                        out_hbm.at[cid, pl.ds(sid * RPS, RPS)])

    return agg_kernel


_RB = 1024
_NB = NPAD // _RB


def _tc_b_body(x_ref, w_ref, deg_ref, hs_ref, dis_ref):
    deg = deg_ref[0, :, 0:1] + deg_ref[1, :, 0:1] + 1.0
    dis = lax.rsqrt(deg)
    h = jnp.dot(x_ref[...], w_ref[...], preferred_element_type=jnp.float32)
    hs_ref[...] = h * dis
    dis_ref[...] = jnp.broadcast_to(dis, (_RB, 16))


def _tc_d_body(acc_ref, hs_ref, dis_ref, b1_ref, w2_ref, out_ref):
    dis = dis_ref[:, 0:1]
    pre = (acc_ref[0] + acc_ref[1] + hs_ref[...]) * dis + b1_ref[...]
    h = jnp.maximum(pre, 0.0)
    out_ref[...] = jnp.dot(h, w2_ref[...],
                           preferred_element_type=jnp.float32) * dis


def _tc_f_body(acc_ref, hs_ref, dis_ref, b2_ref, out_ref):
    out_ref[...] = dis_ref[...] * (acc_ref[0] + acc_ref[1] + hs_ref[...]) \
        + b2_ref[...]


def _tc_b(x_pad, W1, degp):
    return pl.pallas_call(
        _tc_b_body,
        grid=(_NB,),
        in_specs=[
            pl.BlockSpec((_RB, D_IN), lambda i: (i, 0)),
            pl.BlockSpec((D_IN, HID), lambda i: (0, 0)),
            pl.BlockSpec((NC, _RB, 16), lambda i: (0, i, 0)),
        ],
        out_specs=[
            pl.BlockSpec((_RB, HID), lambda i: (i, 0)),
            pl.BlockSpec((_RB, 16), lambda i: (i, 0)),
        ],
        out_shape=[
            jax.ShapeDtypeStruct((NPAD, HID), jnp.float32),
            jax.ShapeDtypeStruct((NPAD, 16), jnp.float32),
        ],
    )(x_pad, W1, degp)


def _tc_d(acc1, hs1, dis16, b1, W2):
    return pl.pallas_call(
        _tc_d_body,
        grid=(_NB,),
        in_specs=[
            pl.BlockSpec((NC, _RB, HID), lambda i: (0, i, 0)),
            pl.BlockSpec((_RB, HID), lambda i: (i, 0)),
            pl.BlockSpec((_RB, 16), lambda i: (i, 0)),
            pl.BlockSpec((1, HID), lambda i: (0, 0)),
            pl.BlockSpec((HID, N_CLS), lambda i: (0, 0)),
        ],
        out_specs=pl.BlockSpec((_RB, N_CLS), lambda i: (i, 0)),
        out_shape=jax.ShapeDtypeStruct((NPAD, N_CLS), jnp.float32),
    )(acc1, hs1, dis16, b1, W2)


def _tc_f(acc2, hs2, dis16, b2):
    return pl.pallas_call(
        _tc_f_body,
        grid=(_NB,),
        in_specs=[
            pl.BlockSpec((NC, _RB, N_CLS), lambda i: (0, i, 0)),
            pl.BlockSpec((_RB, N_CLS), lambda i: (i, 0)),
            pl.BlockSpec((_RB, 16), lambda i: (i, 0)),
            pl.BlockSpec((1, N_CLS), lambda i: (0, 0)),
        ],
        out_specs=pl.BlockSpec((_RB, N_CLS), lambda i: (i, 0)),
        out_shape=jax.ShapeDtypeStruct((NPAD, N_CLS), jnp.float32),
    )(acc2, hs2, dis16, b2)


def kernel(x, edge_index, W1, b1, W2, b2):
    pad_src = jnp.full((EP - E,), NPAD - 2, jnp.int32)
    pad_dst = jnp.full((EP - E,), NPAD - 1, jnp.int32)
    src3d = jnp.concatenate([edge_index[0], pad_src]).reshape(NW, NCH, CH)
    dst3d = jnp.concatenate([edge_index[1], pad_dst]).reshape(NW, NCH, CH)
    x_pad = jnp.pad(x, ((0, NPAD - N), (0, 0)))
    ones16 = jnp.ones((CH, 16), jnp.float32)
    zeros16 = jnp.zeros((NPAD, 16), jnp.float32)
    zeros128 = jnp.zeros((NPAD, HID), jnp.float32)

    degp = _make_deg_kernel()(dst3d, ones16, zeros16)
    hs1, dis16 = _tc_b(x_pad, W1, degp)
    acc1 = _make_agg_kernel(HID)(hs1, src3d, dst3d, zeros128)
    hs2 = _tc_d(acc1, hs1, dis16, b1.reshape(1, HID), W2)
    acc2 = _make_agg_kernel(N_CLS)(hs2, src3d, dst3d, zeros16)
    out = _tc_f(acc2, hs2, dis16, b2.reshape(1, N_CLS))
    return out[:N]


# speedup vs baseline: 1.8167x; 1.8167x over previous
import functools

import jax
import jax.numpy as jnp
from jax import lax
from jax.experimental import pallas as pl
from jax.experimental.pallas import tpu as pltpu
from jax.experimental.pallas import tpu_sc as plsc

N = 10000
E = 320000
D_IN = 128
HID = 128
N_CLS = 16

NPAD = 10240
NC = 2
NS = 16
NW = NC * NS
CH = 128
NCH = 80
EP = NW * NCH * CH
RPS = NPAD // NS


@functools.cache
def _make_deg_kernel():
    mesh = plsc.VectorSubcoreMesh(core_axis_name="c", subcore_axis_name="s")

    @functools.partial(
        pl.kernel,
        mesh=mesh,
        out_type=jax.ShapeDtypeStruct((NC, NPAD, 16), jnp.float32),
        scratch_types=[
            pltpu.VMEM((NCH, CH), jnp.int32),
            pltpu.VMEM((CH, 16), jnp.float32),
            pltpu.VMEM_SHARED((NPAD, 16), jnp.float32),
        ],
        compiler_params=pltpu.CompilerParams(use_tc_tiling_on_sc=False),
    )
    def deg_kernel(dst3d, ones_hbm, zeros_hbm, out_hbm, dst_v, ones_v, acc_s):
        cid = lax.axis_index("c")
        sid = lax.axis_index("s")
        wid = cid * NS + sid
        pltpu.sync_copy(zeros_hbm.at[pl.ds(sid * RPS, RPS)],
                        acc_s.at[pl.ds(sid * RPS, RPS)])
        pltpu.sync_copy(dst3d.at[wid], dst_v)
        pltpu.sync_copy(ones_hbm, ones_v)
        plsc.subcore_barrier()

        def body(j, carry):
            pltpu.sync_copy(ones_v, acc_s.at[dst_v.at[j]], add=True)
            return carry

        lax.fori_loop(0, NCH, body, 0)
        plsc.subcore_barrier()
        pltpu.sync_copy(acc_s.at[pl.ds(sid * RPS, RPS)],
                        out_hbm.at[cid, pl.ds(sid * RPS, RPS)])

    return deg_kernel


@functools.cache
def _make_agg_kernel(D):
    mesh = plsc.VectorSubcoreMesh(core_axis_name="c", subcore_axis_name="s")

    @functools.partial(
        pl.kernel,
        mesh=mesh,
        out_type=jax.ShapeDtypeStruct((NC, NPAD, D), jnp.float32),
        scratch_types=[
            pltpu.VMEM((NCH, CH), jnp.int32),
            pltpu.VMEM((NCH, CH), jnp.int32),
            pltpu.VMEM((CH, D), jnp.float32),
            pltpu.VMEM_SHARED((NPAD, D), jnp.float32),
            pltpu.SemaphoreType.DMA,
        ],
        compiler_params=pltpu.CompilerParams(
            use_tc_tiling_on_sc=(D % 128 == 0)),
    )
    def agg_kernel(hs_hbm, src3d, dst3d, zeros_hbm, out_hbm,
                   src_v, dst_v, rows_v, acc_s, sem):
        cid = lax.axis_index("c")
        sid = lax.axis_index("s")
        wid = cid * NS + sid
        pltpu.sync_copy(zeros_hbm.at[pl.ds(sid * RPS, RPS)],
                        acc_s.at[pl.ds(sid * RPS, RPS)])
        pltpu.sync_copy(src3d.at[wid], src_v)
        pltpu.sync_copy(dst3d.at[wid], dst_v)
        plsc.subcore_barrier()

        def body(j, carry):
            pltpu.async_copy(hs_hbm.at[src_v.at[j]], rows_v, sem).wait()
            pltpu.sync_copy(rows_v, acc_s.at[dst_v.at[j]], add=True)
            return carry

        lax.fori_loop(0, NCH, body, 0)
        plsc.subcore_barrier()
        pltpu.sync_copy(acc_s.at[pl.ds(sid * RPS, RPS)],
                        out_hbm.at[cid, pl.ds(sid * RPS, RPS)])

    return agg_kernel


_RB = 1024
_NB = NPAD // _RB


def _tc_b_body(x_ref, w_ref, deg_ref, hs_ref, dis_ref):
    deg = deg_ref[0, :, 0:1] + deg_ref[1, :, 0:1] + 1.0
    dis = lax.rsqrt(deg)
    h = jnp.dot(x_ref[...], w_ref[...], preferred_element_type=jnp.float32)
    hs_ref[...] = h * dis
    dis_ref[...] = jnp.broadcast_to(dis, (_RB, 16))


def _tc_d_body(acc_ref, hs_ref, dis_ref, b1_ref, w2_ref, out_ref):
    dis = dis_ref[:, 0:1]
    pre = (acc_ref[0] + acc_ref[1] + hs_ref[...]) * dis + b1_ref[...]
    h = jnp.maximum(pre, 0.0)
    out_ref[...] = jnp.dot(h, w2_ref[...],
                           preferred_element_type=jnp.float32) * dis


def _tc_f_body(acc_ref, hs_ref, dis_ref, b2_ref, out_ref):
    out_ref[...] = dis_ref[...] * (acc_ref[0] + acc_ref[1] + hs_ref[...]) \
        + b2_ref[...]


def _tc_b(x_pad, W1, degp):
    return pl.pallas_call(
        _tc_b_body,
        grid=(_NB,),
        in_specs=[
            pl.BlockSpec((_RB, D_IN), lambda i: (i, 0)),
            pl.BlockSpec((D_IN, HID), lambda i: (0, 0)),
            pl.BlockSpec((NC, _RB, 16), lambda i: (0, i, 0)),
        ],
        out_specs=[
            pl.BlockSpec((_RB, HID), lambda i: (i, 0)),
            pl.BlockSpec((_RB, 16), lambda i: (i, 0)),
        ],
        out_shape=[
            jax.ShapeDtypeStruct((NPAD, HID), jnp.float32),
            jax.ShapeDtypeStruct((NPAD, 16), jnp.float32),
        ],
    )(x_pad, W1, degp)


def _tc_d(acc1, hs1, dis16, b1, W2):
    return pl.pallas_call(
        _tc_d_body,
        grid=(_NB,),
        in_specs=[
            pl.BlockSpec((NC, _RB, HID), lambda i: (0, i, 0)),
            pl.BlockSpec((_RB, HID), lambda i: (i, 0)),
            pl.BlockSpec((_RB, 16), lambda i: (i, 0)),
            pl.BlockSpec((1, HID), lambda i: (0, 0)),
            pl.BlockSpec((HID, N_CLS), lambda i: (0, 0)),
        ],
        out_specs=pl.BlockSpec((_RB, N_CLS), lambda i: (i, 0)),
        out_shape=jax.ShapeDtypeStruct((NPAD, N_CLS), jnp.float32),
    )(acc1, hs1, dis16, b1, W2)


def _tc_f(acc2, hs2, dis16, b2):
    return pl.pallas_call(
        _tc_f_body,
        grid=(_NB,),
        in_specs=[
            pl.BlockSpec((NC, _RB, N_CLS), lambda i: (0, i, 0)),
            pl.BlockSpec((_RB, N_CLS), lambda i: (i, 0)),
            pl.BlockSpec((_RB, 16), lambda i: (i, 0)),
            pl.BlockSpec((1, N_CLS), lambda i: (0, 0)),
        ],
        out_specs=pl.BlockSpec((_RB, N_CLS), lambda i: (i, 0)),
        out_shape=jax.ShapeDtypeStruct((NPAD, N_CLS), jnp.float32),
    )(acc2, hs2, dis16, b2)


def kernel(x, edge_index, W1, b1, W2, b2):
    pad_idx = N + jnp.arange(EP - E, dtype=jnp.int32) % (NPAD - N)
    pad_src = pad_idx
    pad_dst = pad_idx
    src3d = jnp.concatenate([edge_index[0], pad_src]).reshape(NW, NCH, CH)
    dst3d = jnp.concatenate([edge_index[1], pad_dst]).reshape(NW, NCH, CH)
    x_pad = jnp.pad(x, ((0, NPAD - N), (0, 0)))
    ones16 = jnp.ones((CH, 16), jnp.float32)
    zeros16 = jnp.zeros((NPAD, 16), jnp.float32)
    zeros128 = jnp.zeros((NPAD, HID), jnp.float32)

    degp = _make_deg_kernel()(dst3d, ones16, zeros16)
    hs1, dis16 = _tc_b(x_pad, W1, degp)
    acc1 = _make_agg_kernel(HID)(hs1, src3d, dst3d, zeros128)
    hs2 = _tc_d(acc1, hs1, dis16, b1.reshape(1, HID), W2)
    acc2 = _make_agg_kernel(N_CLS)(hs2, src3d, dst3d, zeros16)
    out = _tc_f(acc2, hs2, dis16, b2.reshape(1, N_CLS))
    return out[:N]


# speedup vs baseline: 2.2468x; 1.2367x over previous
import functools

import jax
import jax.numpy as jnp
from jax import lax
from jax.experimental import pallas as pl
from jax.experimental.pallas import tpu as pltpu
from jax.experimental.pallas import tpu_sc as plsc

N = 10000
E = 320000
D_IN = 128
HID = 128
N_CLS = 16

NPAD = 10240
NC = 2
NS = 16
NW = NC * NS
CH = 128
NCH = 80
EP = NW * NCH * CH
RPS = NPAD // NS


@functools.cache
def _make_deg_kernel():
    mesh = plsc.VectorSubcoreMesh(core_axis_name="c", subcore_axis_name="s")

    @functools.partial(
        pl.kernel,
        mesh=mesh,
        out_type=jax.ShapeDtypeStruct((NC, NPAD, 16), jnp.float32),
        scratch_types=[
            pltpu.VMEM((NCH, CH), jnp.int32),
            pltpu.VMEM((CH, 16), jnp.float32),
            pltpu.VMEM_SHARED((NPAD, 16), jnp.float32),
        ],
        compiler_params=pltpu.CompilerParams(use_tc_tiling_on_sc=False),
    )
    def deg_kernel(dst3d, ones_hbm, zeros_hbm, out_hbm, dst_v, ones_v, acc_s):
        cid = lax.axis_index("c")
        sid = lax.axis_index("s")
        wid = cid * NS + sid
        pltpu.sync_copy(zeros_hbm.at[pl.ds(sid * RPS, RPS)],
                        acc_s.at[pl.ds(sid * RPS, RPS)])
        pltpu.sync_copy(dst3d.at[wid], dst_v)
        pltpu.sync_copy(ones_hbm, ones_v)
        plsc.subcore_barrier()

        def body(j, carry):
            pltpu.sync_copy(ones_v, acc_s.at[dst_v.at[j]], add=True)
            return carry

        lax.fori_loop(0, NCH, body, 0)
        plsc.subcore_barrier()
        pltpu.sync_copy(acc_s.at[pl.ds(sid * RPS, RPS)],
                        out_hbm.at[cid, pl.ds(sid * RPS, RPS)])

    return deg_kernel


@functools.cache
def _make_agg_kernel(D, NB, CHk, NCHk, tiled):
    mesh = plsc.VectorSubcoreMesh(core_axis_name="c", subcore_axis_name="s")

    @functools.partial(
        pl.kernel,
        mesh=mesh,
        out_type=jax.ShapeDtypeStruct((NC, NPAD, D), jnp.float32),
        scratch_types=[
            pltpu.VMEM((NCHk, CHk), jnp.int32),
            pltpu.VMEM((NCHk, CHk), jnp.int32),
            [pltpu.VMEM((CHk, D), jnp.float32)] * NB,
            [pltpu.SemaphoreType.DMA] * NB,
            [pltpu.SemaphoreType.DMA] * NB,
            pltpu.VMEM_SHARED((NPAD, D), jnp.float32),
        ],
        compiler_params=pltpu.CompilerParams(use_tc_tiling_on_sc=tiled),
    )
    def agg_kernel(hs_hbm, src3d, dst3d, zeros_hbm, out_hbm,
                   src_v, dst_v, rows, sem_g, sem_s, acc_s):
        cid = lax.axis_index("c")
        sid = lax.axis_index("s")
        wid = cid * NS + sid
        pltpu.sync_copy(zeros_hbm.at[pl.ds(sid * RPS, RPS)],
                        acc_s.at[pl.ds(sid * RPS, RPS)])
        pltpu.sync_copy(src3d.at[wid], src_v)
        pltpu.sync_copy(dst3d.at[wid], dst_v)
        plsc.subcore_barrier()

        def g_start(b, j):
            pltpu.make_async_copy(
                hs_hbm.at[src_v.at[j]], rows[b], sem_g[b]).start()

        def g_wait(b):
            pltpu.make_async_copy(
                hs_hbm.at[src_v.at[0]], rows[b], sem_g[b]).wait()

        def s_start(b, j):
            pltpu.make_async_copy(
                rows[b], acc_s.at[dst_v.at[j]], sem_s[b]).start(add=True)

        def s_wait(b):
            pltpu.make_async_copy(
                rows[b], acc_s.at[dst_v.at[0]], sem_s[b]).wait()

        for b in range(NB):
            g_start(b, b)

        def group(gi, carry):
            base = gi * NB
            for b in range(NB):
                g_wait(b)
                s_start(b, base + b)
            for b in range(NB):
                jn = base + b + NB

                @pl.when(jn < NCHk)
                def _(b=b, jn=jn):
                    s_wait(b)
                    g_start(b, jn)

            return carry

        lax.fori_loop(0, NCHk // NB, group, 0)
        for b in range(NB):
            s_wait(b)
        plsc.subcore_barrier()
        pltpu.sync_copy(acc_s.at[pl.ds(sid * RPS, RPS)],
                        out_hbm.at[cid, pl.ds(sid * RPS, RPS)])

    return agg_kernel


_RB = 1024
_NB = NPAD // _RB


def _tc_b_body(x_ref, w_ref, deg_ref, hs_ref, dis_ref):
    deg = deg_ref[0, :, 0:1] + deg_ref[1, :, 0:1] + 1.0
    dis = lax.rsqrt(deg)
    h = jnp.dot(x_ref[...], w_ref[...], preferred_element_type=jnp.float32)
    hs_ref[...] = h * dis
    dis_ref[...] = jnp.broadcast_to(dis, (_RB, 16))


def _tc_d_body(acc_ref, hs_ref, dis_ref, b1_ref, w2_ref, out_ref):
    dis = dis_ref[:, 0:1]
    pre = (acc_ref[0] + acc_ref[1] + hs_ref[...]) * dis + b1_ref[...]
    h = jnp.maximum(pre, 0.0)
    out_ref[...] = jnp.dot(h, w2_ref[...],
                           preferred_element_type=jnp.float32) * dis


def _tc_f_body(acc_ref, hs_ref, dis_ref, b2_ref, out_ref):
    out_ref[...] = dis_ref[...] * (acc_ref[0] + acc_ref[1] + hs_ref[...]) \
        + b2_ref[...]


def _tc_b(x_pad, W1, degp):
    return pl.pallas_call(
        _tc_b_body,
        grid=(_NB,),
        in_specs=[
            pl.BlockSpec((_RB, D_IN), lambda i: (i, 0)),
            pl.BlockSpec((D_IN, HID), lambda i: (0, 0)),
            pl.BlockSpec((NC, _RB, 16), lambda i: (0, i, 0)),
        ],
        out_specs=[
            pl.BlockSpec((_RB, HID), lambda i: (i, 0)),
            pl.BlockSpec((_RB, 16), lambda i: (i, 0)),
        ],
        out_shape=[
            jax.ShapeDtypeStruct((NPAD, HID), jnp.float32),
            jax.ShapeDtypeStruct((NPAD, 16), jnp.float32),
        ],
    )(x_pad, W1, degp)


def _tc_d(acc1, hs1, dis16, b1, W2):
    return pl.pallas_call(
        _tc_d_body,
        grid=(_NB,),
        in_specs=[
            pl.BlockSpec((NC, _RB, HID), lambda i: (0, i, 0)),
            pl.BlockSpec((_RB, HID), lambda i: (i, 0)),
            pl.BlockSpec((_RB, 16), lambda i: (i, 0)),
            pl.BlockSpec((1, HID), lambda i: (0, 0)),
            pl.BlockSpec((HID, N_CLS), lambda i: (0, 0)),
        ],
        out_specs=pl.BlockSpec((_RB, N_CLS), lambda i: (i, 0)),
        out_shape=jax.ShapeDtypeStruct((NPAD, N_CLS), jnp.float32),
    )(acc1, hs1, dis16, b1, W2)


def _tc_f(acc2, hs2, dis16, b2):
    return pl.pallas_call(
        _tc_f_body,
        grid=(_NB,),
        in_specs=[
            pl.BlockSpec((NC, _RB, N_CLS), lambda i: (0, i, 0)),
            pl.BlockSpec((_RB, N_CLS), lambda i: (i, 0)),
            pl.BlockSpec((_RB, 16), lambda i: (i, 0)),
            pl.BlockSpec((1, N_CLS), lambda i: (0, 0)),
        ],
        out_specs=pl.BlockSpec((_RB, N_CLS), lambda i: (i, 0)),
        out_shape=jax.ShapeDtypeStruct((NPAD, N_CLS), jnp.float32),
    )(acc2, hs2, dis16, b2)


def kernel(x, edge_index, W1, b1, W2, b2):
    pad_idx = N + jnp.arange(EP - E, dtype=jnp.int32) % (NPAD - N)
    pad_src = pad_idx
    pad_dst = pad_idx
    EP2 = NW * 108 * 96
    pad2 = N + jnp.arange(EP2 - E, dtype=jnp.int32) % (NPAD - N)
    src3d96 = jnp.concatenate([edge_index[0], pad2]).reshape(NW, 108, 96)
    dst3d96 = jnp.concatenate([edge_index[1], pad2]).reshape(NW, 108, 96)
    src3d = jnp.concatenate([edge_index[0], pad_src]).reshape(NW, NCH, CH)
    dst3d = jnp.concatenate([edge_index[1], pad_dst]).reshape(NW, NCH, CH)
    x_pad = jnp.pad(x, ((0, NPAD - N), (0, 0)))
    ones16 = jnp.ones((CH, 16), jnp.float32)
    zeros16 = jnp.zeros((NPAD, 16), jnp.float32)
    zeros128 = jnp.zeros((NPAD, HID), jnp.float32)

    degp = _make_deg_kernel()(dst3d, ones16, zeros16)
    hs1, dis16 = _tc_b(x_pad, W1, degp)
    acc1 = _make_agg_kernel(HID, 2, 96, 108, False)(
        hs1, src3d96, dst3d96, zeros128)
    hs2 = _tc_d(acc1, hs1, dis16, b1.reshape(1, HID), W2)
    acc2 = _make_agg_kernel(N_CLS, 4, CH, NCH, False)(
        hs2, src3d, dst3d, zeros16)
    out = _tc_f(acc2, hs2, dis16, b2.reshape(1, N_CLS))
    return out[:N]
